# baseline (speedup 1.0000x reference)
import functools

import jax
import jax.numpy as jnp
from jax import lax
from jax.experimental import pallas as pl
from jax.experimental.pallas import tpu as pltpu
from jax.experimental.pallas import tpu_sc as plsc

N = 50000
E = 800000
D = 64
DE = 4

BN = 2000
BE = 2048

CHUNK = 128
NCHUNKS = E // CHUNK
NW = 32
NT = 16
NROWS_TAB = 50176


EPW = E // NW
GFULL = EPW // CHUNK
GTAIL = EPW - GFULL * CHUNK


def _sc_gather_body(dst_hbm, src_hbm, ab_hbm, ba_hbm, g_hbm,
                    dsti, srci, buf, dstt, srct, tbuf, sema, semb):
    cid = lax.axis_index("c")
    sid = lax.axis_index("s")
    wid = sid * 2 + cid
    base = wid * EPW

    def body(j, _):
        e0 = base + j * CHUNK
        pltpu.sync_copy(dst_hbm.at[pl.ds(e0, CHUNK)], dsti)
        pltpu.sync_copy(src_hbm.at[pl.ds(e0, CHUNK)], srci)
        pltpu.async_copy(ab_hbm.at[dsti], buf, sema).wait()
        pltpu.async_copy(ba_hbm.at[srci], buf, semb, add=True).wait()
        pltpu.sync_copy(buf, g_hbm.at[pl.ds(e0, CHUNK)])
        return _

    lax.fori_loop(0, GFULL, body, None)

    e0 = base + GFULL * CHUNK
    pltpu.sync_copy(dst_hbm.at[pl.ds(e0, GTAIL)], dstt)
    pltpu.sync_copy(src_hbm.at[pl.ds(e0, GTAIL)], srct)
    pltpu.async_copy(ab_hbm.at[dstt], tbuf, sema).wait()
    pltpu.async_copy(ba_hbm.at[srct], tbuf, semb, add=True).wait()
    pltpu.sync_copy(tbuf, g_hbm.at[pl.ds(e0, GTAIL)])


def _sc_gather(dst, src, ab_tab, ba_tab):
    mesh = plsc.VectorSubcoreMesh(core_axis_name="c", subcore_axis_name="s")
    f = pl.kernel(
        _sc_gather_body,
        out_type=jax.ShapeDtypeStruct((E_PAD, 2 * D), jnp.float32),
        mesh=mesh,
        scratch_types=[
            pltpu.VMEM((CHUNK,), jnp.int32),
            pltpu.VMEM((CHUNK,), jnp.int32),
            pltpu.VMEM((CHUNK, 2 * D), jnp.float32),
            pltpu.VMEM((GTAIL,), jnp.int32),
            pltpu.VMEM((GTAIL,), jnp.int32),
            pltpu.VMEM((GTAIL, 2 * D), jnp.float32),
            pltpu.SemaphoreType.DMA,
            pltpu.SemaphoreType.DMA,
        ],
    )
    return f(dst, src, ab_tab, ba_tab)


NGRP = 16
DGC = 4
NRR = 2
N_PAD = 51200
RSPAN = N_PAD // NRR
E_PAD = 802816
CHUNK_S = 2048
TR_C = CHUNK_S // 128
NCH_S = 390
TAIL_S = E - NCH_S * CHUNK_S
TRT = TAIL_S // 128


def _sc_scatter_body(dst_hbm, mt_hbm, aggr_hbm,
                     dsti0, dsti1, cb0, cb1, table, bounce, sem0, sem1):
    cid = lax.axis_index("c")
    sid = lax.axis_index("s")
    wid = sid * 2 + cid
    r = wid // NGRP
    g = wid - r * NGRP
    base_row = r * RSPAN
    c0 = g * DGC

    zeros16 = jnp.zeros((16,), jnp.float32)

    def zb(i, _):
        for u in range(4):
            table[pl.ds(i * 64 + u * 16, 16)] = zeros16
        return _

    lax.fori_loop(0, DGC * RSPAN // 64, zb, None)

    def fire(j, ibuf, dbuf, sem):
        b0 = j * TR_C
        pltpu.async_copy(dst_hbm.at[pl.ds(j * CHUNK_S, CHUNK_S)], ibuf, sem)
        for c in range(DGC):
            pltpu.async_copy(mt_hbm.at[c0 + c, pl.ds(b0, TR_C)],
                             dbuf.at[c], sem)

    def drain(j, ibuf, dbuf, sem):
        b0 = j * TR_C
        pltpu.make_async_copy(dst_hbm.at[pl.ds(j * CHUNK_S, CHUNK_S)],
                              ibuf, sem).wait()
        for c in range(DGC):
            pltpu.make_async_copy(mt_hbm.at[c0 + c, pl.ds(b0, TR_C)],
                                  dbuf.at[c], sem).wait()

    def process(ibuf, dbuf, ntr):
        def rrow(rr, _):
            for u in range(8):
                o = u * 16
                d = ibuf[pl.ds(rr * 128 + o, 16)]
                local = d - base_row
                msk = (local >= 0) & (local < RSPAN)
                sel = jnp.minimum(jnp.maximum(local, 0), RSPAN - 1)
                for c in range(DGC):
                    val = dbuf[c, rr, pl.ds(o, 16)]
                    plsc.addupdate_scatter(table, [sel + c * RSPAN], val,
                                           mask=msk)
            return _

        lax.fori_loop(0, ntr, rrow, None)

    fire(0, dsti0, cb0, sem0)
    fire(1, dsti1, cb1, sem1)

    def body(t, _):
        drain(2 * t, dsti0, cb0, sem0)
        process(dsti0, cb0, TR_C)
        fire(2 * t + 2, dsti0, cb0, sem0)
        drain(2 * t + 1, dsti1, cb1, sem1)
        process(dsti1, cb1, TR_C)
        fire(2 * t + 3, dsti1, cb1, sem1)
        return _

    lax.fori_loop(0, NCH_S // 2, body, None)
    drain(390, dsti0, cb0, sem0)
    process(dsti0, cb0, TRT)
    drain(391, dsti1, cb1, sem1)

    for c in range(DGC):
        for kk in range(RSPAN // 1024):
            base = c * RSPAN + kk * 1024

            def cp(rr, _):
                for u in range(8):
                    o = u * 16
                    bounce[rr, pl.ds(o, 16)] = table[
                        pl.ds(base + rr * 128 + o, 16)]
                return _

            lax.fori_loop(0, 8, cp, None)
            pltpu.sync_copy(
                bounce,
                aggr_hbm.at[c0 + c, pl.ds(r * (RSPAN // 128) + kk * 8, 8)])


def _sc_scatter(dst_pad, mt):
    mesh = plsc.VectorSubcoreMesh(core_axis_name="c", subcore_axis_name="s")
    f = pl.kernel(
        _sc_scatter_body,
        out_type=jax.ShapeDtypeStruct((D, N_PAD // 128, 128), jnp.float32),
        mesh=mesh,
        scratch_types=[
            pltpu.VMEM((CHUNK_S,), jnp.int32),
            pltpu.VMEM((CHUNK_S,), jnp.int32),
            pltpu.VMEM((DGC, TR_C, 128), jnp.float32),
            pltpu.VMEM((DGC, TR_C, 128), jnp.float32),
            pltpu.VMEM((DGC * RSPAN,), jnp.float32),
            pltpu.VMEM((8, 128), jnp.float32),
            pltpu.SemaphoreType.DMA,
            pltpu.SemaphoreType.DMA,
        ],
    )
    return f(dst_pad, mt)


def _node_pre_body(h_ref, w_ref, ab_ref, ba_ref):
    ab = jnp.dot(h_ref[...], w_ref[...], preferred_element_type=jnp.float32)
    ab_ref[...] = ab
    ba_ref[...] = jnp.concatenate([ab[:, D:], ab[:, :D]], axis=1)


def _node_pre(h, w_ij_t):
    return pl.pallas_call(
        _node_pre_body,
        grid=(N // BN,),
        in_specs=[
            pl.BlockSpec((BN, D), lambda i: (i, 0)),
            pl.BlockSpec((D, 2 * D), lambda i: (0, 0)),
        ],
        out_specs=[pl.BlockSpec((BN, 2 * D), lambda i: (i, 0)),
                   pl.BlockSpec((BN, 2 * D), lambda i: (i, 0))],
        out_shape=[jax.ShapeDtypeStruct((N, 2 * D), jnp.float32),
                   jax.ShapeDtypeStruct((N, 2 * D), jnp.float32)],
    )(h, w_ij_t)


def _edge_mlp_body(g_ref, ea_ref, w1e_ref, b1_ref, w2_ref, b2_ref,
                   m_ref):
    c = jnp.dot(ea_ref[...], w1e_ref[...], preferred_element_type=jnp.float32)
    t = jnp.tanh(g_ref[:, :D] + c + b1_ref[...])
    m = jnp.tanh(jnp.dot(t, w2_ref[...], preferred_element_type=jnp.float32)
                 + b2_ref[...])
    row = pl.program_id(0) * BE + lax.broadcasted_iota(jnp.int32, (BE, D), 0)
    m_ref[...] = jnp.where(row < E, m, 0.0)


def _edge_mlp(g, edge_attr, w1e_t, b1, w2_t, b2):
    return pl.pallas_call(
        _edge_mlp_body,
        grid=(E_PAD // BE,),
        in_specs=[
            pl.BlockSpec((BE, 2 * D), lambda i: (i, 0)),
            pl.BlockSpec((BE, DE), lambda i: (i, 0)),
            pl.BlockSpec((DE, D), lambda i: (0, 0)),
            pl.BlockSpec((1, D), lambda i: (0, 0)),
            pl.BlockSpec((D, D), lambda i: (0, 0)),
            pl.BlockSpec((1, D), lambda i: (0, 0)),
        ],
        out_specs=pl.BlockSpec((BE, D), lambda i: (i, 0)),
        out_shape=jax.ShapeDtypeStruct((E_PAD, D), jnp.float32),
    )(g, edge_attr, w1e_t, b1, w2_t, b2)


def _node_mlp_body(h_ref, a_ref, wh_ref, wa_ref, b1_ref, w2_ref,
                   b2_ref, u_ref):
    acc = jnp.dot(h_ref[...], wh_ref[...],
                  preferred_element_type=jnp.float32)
    acc = acc + jnp.dot(a_ref[...], wa_ref[...],
                        preferred_element_type=jnp.float32)
    t = jnp.tanh(acc + b1_ref[...])
    u_ref[...] = jnp.tanh(
        jnp.dot(t, w2_ref[...], preferred_element_type=jnp.float32)
        + b2_ref[...])


def _node_mlp(h, aggr4, wh_t, wa_t, bu1, wu2_t, bu2):
    return pl.pallas_call(
        _node_mlp_body,
        grid=(N // BN,),
        in_specs=[
            pl.BlockSpec((BN, D), lambda i: (i, 0)),
            pl.BlockSpec((BN, D), lambda i: (i, 0)),
            pl.BlockSpec((D, D), lambda i: (0, 0)),
            pl.BlockSpec((D, D), lambda i: (0, 0)),
            pl.BlockSpec((1, D), lambda i: (0, 0)),
            pl.BlockSpec((D, D), lambda i: (0, 0)),
            pl.BlockSpec((1, D), lambda i: (0, 0)),
        ],
        out_specs=pl.BlockSpec((BN, D), lambda i: (i, 0)),
        out_shape=jax.ShapeDtypeStruct((N, D), jnp.float32),
    )(h, aggr4, wh_t, wa_t, bu1, wu2_t, bu2)


def kernel(h, edge_index, edge_attr, W_msg1, b_msg1, W_msg2, b_msg2,
           W_upd1, b_upd1, W_upd2, b_upd2):
    src = edge_index[0]
    dst = edge_index[1]

    w_ij_t = jnp.concatenate([W_msg1[:, :D].T, W_msg1[:, D:2 * D].T], axis=1)
    ab, ba = _node_pre(h, w_ij_t)

    g = _sc_gather(dst, src, ab, ba)

    ea_pad = jnp.pad(edge_attr, ((0, E_PAD - E), (0, 0)))
    m3 = _edge_mlp(g, ea_pad, W_msg1[:, 2 * D:].T, b_msg1[None, :],
                   W_msg2.T, b_msg2[None, :])

    dst_pad = jnp.pad(dst, (0, E_PAD - E))
    aggr = jnp.zeros((N, D), jnp.float32).at[dst_pad].add(
        m3, mode="promise_in_bounds")

    u = _node_mlp(h, aggr, W_upd1[:, :D].T, W_upd1[:, D:].T,
                  b_upd1[None, :], W_upd2.T, b_upd2[None, :])
    return u


# speedup vs baseline: 1.0652x; 1.0652x over previous
import functools

import jax
import jax.numpy as jnp
from jax import lax
from jax.experimental import pallas as pl
from jax.experimental.pallas import tpu as pltpu
from jax.experimental.pallas import tpu_sc as plsc

N = 50000
E = 800000
D = 64
DE = 4

BN = 2000
BE = 2048

CHUNK = 128
NCHUNKS = E // CHUNK
NW = 32
NT = 16
NROWS_TAB = 50176


EPW = E // NW
GFULL = EPW // CHUNK
GTAIL = EPW - GFULL * CHUNK


def _sc_gather_body(dst_hbm, src_hbm, ab_hbm, ba_hbm, g_hbm,
                    dsti0, srci0, buf0, dsti1, srci1, buf1,
                    dstt, srct, tbuf, semi, semg, sema, semw):
    cid = lax.axis_index("c")
    sid = lax.axis_index("s")
    wid = sid * 2 + cid
    base = wid * EPW

    def pair(t, _):
        e0 = base + (2 * t) * CHUNK
        e1 = base + (2 * t + 1) * CHUNK
        i0 = pltpu.async_copy(dst_hbm.at[pl.ds(e0, CHUNK)], dsti0, semi)
        i1 = pltpu.async_copy(src_hbm.at[pl.ds(e0, CHUNK)], srci0, semi)
        i2 = pltpu.async_copy(dst_hbm.at[pl.ds(e1, CHUNK)], dsti1, semi)
        i3 = pltpu.async_copy(src_hbm.at[pl.ds(e1, CHUNK)], srci1, semi)
        i0.wait(); i1.wait(); i2.wait(); i3.wait()
        g0 = pltpu.async_copy(ab_hbm.at[dsti0], buf0, semg)
        g1 = pltpu.async_copy(ab_hbm.at[dsti1], buf1, semg)
        g0.wait(); g1.wait()
        a0 = pltpu.async_copy(ba_hbm.at[srci0], buf0, sema, add=True)
        a1 = pltpu.async_copy(ba_hbm.at[srci1], buf1, sema, add=True)
        a0.wait(); a1.wait()
        w0 = pltpu.async_copy(buf0, g_hbm.at[pl.ds(e0, CHUNK)], semw)
        w1 = pltpu.async_copy(buf1, g_hbm.at[pl.ds(e1, CHUNK)], semw)
        w0.wait(); w1.wait()
        return _

    lax.fori_loop(0, GFULL // 2, pair, None)

    e0 = base + (GFULL - 1) * CHUNK
    pltpu.sync_copy(dst_hbm.at[pl.ds(e0, CHUNK)], dsti0)
    pltpu.sync_copy(src_hbm.at[pl.ds(e0, CHUNK)], srci0)
    pltpu.async_copy(ab_hbm.at[dsti0], buf0, semg).wait()
    pltpu.async_copy(ba_hbm.at[srci0], buf0, sema, add=True).wait()
    pltpu.sync_copy(buf0, g_hbm.at[pl.ds(e0, CHUNK)])

    e0 = base + GFULL * CHUNK
    pltpu.sync_copy(dst_hbm.at[pl.ds(e0, GTAIL)], dstt)
    pltpu.sync_copy(src_hbm.at[pl.ds(e0, GTAIL)], srct)
    pltpu.async_copy(ab_hbm.at[dstt], tbuf, semg).wait()
    pltpu.async_copy(ba_hbm.at[srct], tbuf, sema, add=True).wait()
    pltpu.sync_copy(tbuf, g_hbm.at[pl.ds(e0, GTAIL)])


def _sc_gather(dst, src, ab_tab, ba_tab):
    mesh = plsc.VectorSubcoreMesh(core_axis_name="c", subcore_axis_name="s")
    f = pl.kernel(
        _sc_gather_body,
        out_type=jax.ShapeDtypeStruct((E_PAD, 2 * D), jnp.float32),
        mesh=mesh,
        scratch_types=[
            pltpu.VMEM((CHUNK,), jnp.int32),
            pltpu.VMEM((CHUNK,), jnp.int32),
            pltpu.VMEM((CHUNK, 2 * D), jnp.float32),
            pltpu.VMEM((CHUNK,), jnp.int32),
            pltpu.VMEM((CHUNK,), jnp.int32),
            pltpu.VMEM((CHUNK, 2 * D), jnp.float32),
            pltpu.VMEM((GTAIL,), jnp.int32),
            pltpu.VMEM((GTAIL,), jnp.int32),
            pltpu.VMEM((GTAIL, 2 * D), jnp.float32),
            pltpu.SemaphoreType.DMA,
            pltpu.SemaphoreType.DMA,
            pltpu.SemaphoreType.DMA,
            pltpu.SemaphoreType.DMA,
        ],
    )
    return f(dst, src, ab_tab, ba_tab)


NGRP = 16
DGC = 4
NRR = 2
N_PAD = 51200
RSPAN = N_PAD // NRR
E_PAD = 802816
CHUNK_S = 2048
TR_C = CHUNK_S // 128
NCH_S = 390
TAIL_S = E - NCH_S * CHUNK_S
TRT = TAIL_S // 128


def _sc_scatter_body(dst_hbm, mt_hbm, aggr_hbm,
                     dsti0, dsti1, cb0, cb1, table, bounce, sem0, sem1):
    cid = lax.axis_index("c")
    sid = lax.axis_index("s")
    wid = sid * 2 + cid
    r = wid // NGRP
    g = wid - r * NGRP
    base_row = r * RSPAN
    c0 = g * DGC

    zeros16 = jnp.zeros((16,), jnp.float32)

    def zb(i, _):
        for u in range(4):
            table[pl.ds(i * 64 + u * 16, 16)] = zeros16
        return _

    lax.fori_loop(0, DGC * RSPAN // 64, zb, None)

    def fire(j, ibuf, dbuf, sem):
        b0 = j * TR_C
        pltpu.async_copy(dst_hbm.at[pl.ds(j * CHUNK_S, CHUNK_S)], ibuf, sem)
        for c in range(DGC):
            pltpu.async_copy(mt_hbm.at[c0 + c, pl.ds(b0, TR_C)],
                             dbuf.at[c], sem)

    def drain(j, ibuf, dbuf, sem):
        b0 = j * TR_C
        pltpu.make_async_copy(dst_hbm.at[pl.ds(j * CHUNK_S, CHUNK_S)],
                              ibuf, sem).wait()
        for c in range(DGC):
            pltpu.make_async_copy(mt_hbm.at[c0 + c, pl.ds(b0, TR_C)],
                                  dbuf.at[c], sem).wait()

    def process(ibuf, dbuf, ntr):
        def rrow(rr, _):
            for u in range(8):
                o = u * 16
                d = ibuf[pl.ds(rr * 128 + o, 16)]
                local = d - base_row
                msk = (local >= 0) & (local < RSPAN)
                sel = jnp.minimum(jnp.maximum(local, 0), RSPAN - 1)
                for c in range(DGC):
                    val = dbuf[c, rr, pl.ds(o, 16)]
                    plsc.addupdate_scatter(table, [sel + c * RSPAN], val,
                                           mask=msk)
            return _

        lax.fori_loop(0, ntr, rrow, None)

    fire(0, dsti0, cb0, sem0)
    fire(1, dsti1, cb1, sem1)

    def body(t, _):
        drain(2 * t, dsti0, cb0, sem0)
        process(dsti0, cb0, TR_C)
        fire(2 * t + 2, dsti0, cb0, sem0)
        drain(2 * t + 1, dsti1, cb1, sem1)
        process(dsti1, cb1, TR_C)
        fire(2 * t + 3, dsti1, cb1, sem1)
        return _

    lax.fori_loop(0, NCH_S // 2, body, None)
    drain(390, dsti0, cb0, sem0)
    process(dsti0, cb0, TRT)
    drain(391, dsti1, cb1, sem1)

    for c in range(DGC):
        for kk in range(RSPAN // 1024):
            base = c * RSPAN + kk * 1024

            def cp(rr, _):
                for u in range(8):
                    o = u * 16
                    bounce[rr, pl.ds(o, 16)] = table[
                        pl.ds(base + rr * 128 + o, 16)]
                return _

            lax.fori_loop(0, 8, cp, None)
            pltpu.sync_copy(
                bounce,
                aggr_hbm.at[c0 + c, pl.ds(r * (RSPAN // 128) + kk * 8, 8)])


def _sc_scatter(dst_pad, mt):
    mesh = plsc.VectorSubcoreMesh(core_axis_name="c", subcore_axis_name="s")
    f = pl.kernel(
        _sc_scatter_body,
        out_type=jax.ShapeDtypeStruct((D, N_PAD // 128, 128), jnp.float32),
        mesh=mesh,
        scratch_types=[
            pltpu.VMEM((CHUNK_S,), jnp.int32),
            pltpu.VMEM((CHUNK_S,), jnp.int32),
            pltpu.VMEM((DGC, TR_C, 128), jnp.float32),
            pltpu.VMEM((DGC, TR_C, 128), jnp.float32),
            pltpu.VMEM((DGC * RSPAN,), jnp.float32),
            pltpu.VMEM((8, 128), jnp.float32),
            pltpu.SemaphoreType.DMA,
            pltpu.SemaphoreType.DMA,
        ],
    )
    return f(dst_pad, mt)


def _node_pre_body(h_ref, w_ref, ab_ref, ba_ref):
    ab = jnp.dot(h_ref[...], w_ref[...], preferred_element_type=jnp.float32)
    ab_ref[...] = ab
    ba_ref[...] = jnp.concatenate([ab[:, D:], ab[:, :D]], axis=1)


def _node_pre(h, w_ij_t):
    return pl.pallas_call(
        _node_pre_body,
        grid=(N // BN,),
        in_specs=[
            pl.BlockSpec((BN, D), lambda i: (i, 0)),
            pl.BlockSpec((D, 2 * D), lambda i: (0, 0)),
        ],
        out_specs=[pl.BlockSpec((BN, 2 * D), lambda i: (i, 0)),
                   pl.BlockSpec((BN, 2 * D), lambda i: (i, 0))],
        out_shape=[jax.ShapeDtypeStruct((N, 2 * D), jnp.float32),
                   jax.ShapeDtypeStruct((N, 2 * D), jnp.float32)],
    )(h, w_ij_t)


def _edge_mlp_body(g_ref, ea_ref, w1e_ref, b1_ref, w2_ref, b2_ref,
                   m_ref):
    c = jnp.dot(ea_ref[...], w1e_ref[...], preferred_element_type=jnp.float32)
    t = jnp.tanh(g_ref[:, :D] + c + b1_ref[...])
    m = jnp.tanh(jnp.dot(t, w2_ref[...], preferred_element_type=jnp.float32)
                 + b2_ref[...])
    row = pl.program_id(0) * BE + lax.broadcasted_iota(jnp.int32, (BE, D), 0)
    m_ref[...] = jnp.where(row < E, m, 0.0)


def _edge_mlp(g, edge_attr, w1e_t, b1, w2_t, b2):
    return pl.pallas_call(
        _edge_mlp_body,
        grid=(E_PAD // BE,),
        in_specs=[
            pl.BlockSpec((BE, 2 * D), lambda i: (i, 0)),
            pl.BlockSpec((BE, DE), lambda i: (i, 0)),
            pl.BlockSpec((DE, D), lambda i: (0, 0)),
            pl.BlockSpec((1, D), lambda i: (0, 0)),
            pl.BlockSpec((D, D), lambda i: (0, 0)),
            pl.BlockSpec((1, D), lambda i: (0, 0)),
        ],
        out_specs=pl.BlockSpec((BE, D), lambda i: (i, 0)),
        out_shape=jax.ShapeDtypeStruct((E_PAD, D), jnp.float32),
    )(g, edge_attr, w1e_t, b1, w2_t, b2)


def _node_mlp_body(h_ref, a_ref, wh_ref, wa_ref, b1_ref, w2_ref,
                   b2_ref, u_ref):
    acc = jnp.dot(h_ref[...], wh_ref[...],
                  preferred_element_type=jnp.float32)
    acc = acc + jnp.dot(a_ref[...], wa_ref[...],
                        preferred_element_type=jnp.float32)
    t = jnp.tanh(acc + b1_ref[...])
    u_ref[...] = jnp.tanh(
        jnp.dot(t, w2_ref[...], preferred_element_type=jnp.float32)
        + b2_ref[...])


def _node_mlp(h, aggr4, wh_t, wa_t, bu1, wu2_t, bu2):
    return pl.pallas_call(
        _node_mlp_body,
        grid=(N // BN,),
        in_specs=[
            pl.BlockSpec((BN, D), lambda i: (i, 0)),
            pl.BlockSpec((BN, D), lambda i: (i, 0)),
            pl.BlockSpec((D, D), lambda i: (0, 0)),
            pl.BlockSpec((D, D), lambda i: (0, 0)),
            pl.BlockSpec((1, D), lambda i: (0, 0)),
            pl.BlockSpec((D, D), lambda i: (0, 0)),
            pl.BlockSpec((1, D), lambda i: (0, 0)),
        ],
        out_specs=pl.BlockSpec((BN, D), lambda i: (i, 0)),
        out_shape=jax.ShapeDtypeStruct((N, D), jnp.float32),
    )(h, aggr4, wh_t, wa_t, bu1, wu2_t, bu2)


def kernel(h, edge_index, edge_attr, W_msg1, b_msg1, W_msg2, b_msg2,
           W_upd1, b_upd1, W_upd2, b_upd2):
    src = edge_index[0]
    dst = edge_index[1]

    w_ij_t = jnp.concatenate([W_msg1[:, :D].T, W_msg1[:, D:2 * D].T], axis=1)
    ab, ba = _node_pre(h, w_ij_t)

    g = _sc_gather(dst, src, ab, ba)

    ea_pad = jnp.pad(edge_attr, ((0, E_PAD - E), (0, 0)))
    m3 = _edge_mlp(g, ea_pad, W_msg1[:, 2 * D:].T, b_msg1[None, :],
                   W_msg2.T, b_msg2[None, :])

    dst_pad = jnp.pad(dst, (0, E_PAD - E))
    aggr = jnp.zeros((N, D), jnp.float32).at[dst_pad].add(
        m3, mode="promise_in_bounds")

    u = _node_mlp(h, aggr, W_upd1[:, :D].T, W_upd1[:, D:].T,
                  b_upd1[None, :], W_upd2.T, b_upd2[None, :])
    return u
